# baseline (speedup 1.0000x reference)
import functools
import jax
import jax.numpy as jnp
from jax import lax
from jax.experimental import pallas as pl
from jax.experimental.pallas import tpu as pltpu
from jax.experimental.pallas import tpu_sc as plsc


def kernel(x, spatial_emb):
    b, c, h, w = x.shape
    g = spatial_emb.shape[1]
    ch, cw = h // g, w // g
    band = ch * w
    k = 1
    while (k * band) % 128:
        k += 1
    lanes = k * band
    nrg = g // k
    kg = k * g
    tab_rows = c * nrg
    rows_total = b * tab_rows
    nchunk = lanes // 16

    info = plsc.get_sparse_core_info()
    nc, ns = info.num_cores, info.num_subcores
    nw = nc * ns
    rpw = rows_total // nw
    CH = 8
    nck = rpw // CH

    tab = jnp.transpose(spatial_emb[0], (2, 0, 1)).reshape(tab_rows * kg)
    l = jnp.arange(lanes, dtype=jnp.int32)
    code = (l // band) * g + (l % w) // cw
    x2 = x.reshape(rows_total, lanes)

    mesh = plsc.VectorSubcoreMesh(core_axis_name="c", subcore_axis_name="s")

    @functools.partial(
        pl.kernel,
        out_type=jax.ShapeDtypeStruct((rows_total, lanes), jnp.float32),
        mesh=mesh,
        scratch_types=(
            [pltpu.VMEM((rpw * kg,), jnp.float32),
             pltpu.VMEM((lanes,), jnp.int32)]
            + [pltpu.VMEM((CH, lanes), jnp.float32) for _ in range(2)]
            + [pltpu.SemaphoreType.DMA for _ in range(4)]
        ),
        compiler_params=pltpu.CompilerParams(
            needs_layout_passes=False, use_tc_tiling_on_sc=True),
    )
    def sc_add(x_hbm, tab_hbm, code_hbm, out_hbm,
               tab_v, code_v, buf0, buf1, si0, si1, so0, so1):
        wid = lax.axis_index("s") * nc + lax.axis_index("c")
        pltpu.sync_copy(code_hbm, code_v)
        lrow0 = lax.rem(wid * rpw, tab_rows)
        pltpu.sync_copy(tab_hbm.at[pl.ds(lrow0 * kg, rpw * kg)], tab_v)
        row0 = wid * rpw

        bufs = [buf0, buf1]
        sin = [si0, si1]
        sout = [so0, so1]

        def in_copy(gi, s):
            src = x_hbm.at[pl.ds(row0 + gi * CH, CH), :]
            return pltpu.make_async_copy(src, bufs[s], sin[s])

        def out_copy(gi, s):
            dst = out_hbm.at[pl.ds(row0 + gi * CH, CH), :]
            return pltpu.make_async_copy(bufs[s], dst, sout[s])

        def compute_rows(buf, gi, r_lo, r_hi):
            for rin in range(r_lo, r_hi):
                ibase = (gi * CH + rin) * kg

                @plsc.parallel_loop(0, nchunk, unroll=4)
                def _(j):
                    off = j * 16
                    idx = code_v[pl.ds(off, 16)] + ibase
                    ev = plsc.load_gather(tab_v, [idx])
                    buf[rin, pl.ds(off, 16)] = buf[rin, pl.ds(off, 16)] + ev

        in_copy(0, 0).start()
        for gi in range(nck):
            s = gi & 1
            in_copy(gi, s).wait()
            buf = bufs[s]
            compute_rows(buf, gi, 0, CH // 2)
            if gi + 1 < nck:
                if gi >= 1:
                    out_copy(gi - 1, 1 - s).wait()
                in_copy(gi + 1, 1 - s).start()
            compute_rows(buf, gi, CH // 2, CH)
            out_copy(gi, s).start()

        for gi in (nck - 2, nck - 1):
            out_copy(gi, gi & 1).wait()

    out = sc_add(x2, tab, code)
    return out.reshape(b, c, h, w)


# speedup vs baseline: 1.2012x; 1.2012x over previous
import jax
import jax.numpy as jnp
from jax.experimental import pallas as pl
from jax.experimental.pallas import tpu as pltpu


def kernel(x, spatial_emb):
    b, c, h, w = x.shape
    g = spatial_emb.shape[1]
    ch, cw = h // g, w // g
    band = ch * w
    k = 1
    while (k * band) % 128:
        k += 1
    lanes = k * band
    nrg = g // k
    kg = k * g
    rows_total = b * c * nrg

    tab = jnp.transpose(spatial_emb[0], (2, 0, 1)).reshape(c * nrg, kg)
    x2 = x.reshape(rows_total, lanes)

    BR = 256
    nblocks = rows_total // BR
    per_b = c * nrg // BR

    def body(tab_ref, x_ref, o_ref, sel_ref):
        @pl.when(pl.program_id(0) == 0)
        def _():
            l = jax.lax.broadcasted_iota(jnp.int32, (1, lanes), 1)
            code = (l // band) * g + (l % w) // cw
            gg = jax.lax.broadcasted_iota(jnp.int32, (kg, lanes), 0)
            sel_ref[...] = (code == gg).astype(jnp.float32)
        rows = jnp.dot(tab_ref[...], sel_ref[...],
                       preferred_element_type=jnp.float32)
        o_ref[...] = x_ref[...] + rows

    out = pl.pallas_call(
        body,
        grid=(nblocks,),
        in_specs=[
            pl.BlockSpec((BR, kg), lambda i: (i % per_b, 0)),
            pl.BlockSpec((BR, lanes), lambda i: (i, 0)),
        ],
        out_specs=pl.BlockSpec((BR, lanes), lambda i: (i, 0)),
        out_shape=jax.ShapeDtypeStruct((rows_total, lanes), x.dtype),
        scratch_shapes=[pltpu.VMEM((kg, lanes), jnp.float32)],
    )(tab, x2)
    return out.reshape(b, c, h, w)


# speedup vs baseline: 1.2086x; 1.0062x over previous
import jax
import jax.numpy as jnp
from jax.experimental import pallas as pl
from jax.experimental.pallas import tpu as pltpu


def kernel(x, spatial_emb):
    b, c, h, w = x.shape
    g = spatial_emb.shape[1]
    ch, cw = h // g, w // g
    band = ch * w
    k = 1
    while (k * band) % 128:
        k += 1
    lanes = k * band
    nrg = g // k
    kg = k * g
    rows_total = b * c * nrg

    tab = jnp.transpose(spatial_emb[0], (2, 0, 1)).reshape(c * nrg, kg)
    x2 = x.reshape(rows_total, lanes)

    BR = 512
    nblocks = rows_total // BR
    per_b = c * nrg // BR

    def body(tab_ref, x_ref, o_ref, sel_ref):
        @pl.when(pl.program_id(0) == 0)
        def _():
            l = jax.lax.broadcasted_iota(jnp.int32, (1, lanes), 1)
            code = (l // band) * g + (l % w) // cw
            gg = jax.lax.broadcasted_iota(jnp.int32, (kg, lanes), 0)
            sel_ref[...] = (code == gg).astype(jnp.float32)
        rows = jnp.dot(tab_ref[...], sel_ref[...],
                       preferred_element_type=jnp.float32)
        o_ref[...] = x_ref[...] + rows

    out = pl.pallas_call(
        body,
        grid=(nblocks,),
        in_specs=[
            pl.BlockSpec((BR, kg), lambda i: (i % per_b, 0)),
            pl.BlockSpec((BR, lanes), lambda i: (i, 0)),
        ],
        out_specs=pl.BlockSpec((BR, lanes), lambda i: (i, 0)),
        out_shape=jax.ShapeDtypeStruct((rows_total, lanes), x.dtype),
        scratch_shapes=[pltpu.VMEM((kg, lanes), jnp.float32)],
    )(tab, x2)
    return out.reshape(b, c, h, w)
